# baseline (speedup 1.0000x reference)
import functools

import jax
import jax.numpy as jnp
from jax import lax
from jax.experimental import pallas as pl
from jax.experimental.pallas import tpu as pltpu
from jax.experimental.pallas import tpu_sc as plsc

GAMMA = 0.99

B = 16384
A = 1024
RB = 512
NBLK = B // RB

NC = 2
NS = 16
NW = NC * NS
PER_W = B // NW
CH = PER_W // 128
L = 16


def _argmax_body(av_ref, out_ref):
    i = pl.program_id(0)
    av = av_ref[...]
    mx = jnp.max(av, axis=1, keepdims=True)
    rows = lax.broadcasted_iota(jnp.int32, (RB, A), 0) + i * RB
    cols = lax.broadcasted_iota(jnp.int32, (RB, A), 1)
    flat = rows * A + cols
    big = jnp.int32(2**30)
    cand = jnp.where(av == mx, flat, big)
    out_ref[0, 0, :] = jnp.min(cand, axis=1)


def _argmax_flat(action_values):
    out = pl.pallas_call(
        _argmax_body,
        grid=(NBLK,),
        in_specs=[pl.BlockSpec((RB, A), lambda i: (i, 0))],
        out_specs=pl.BlockSpec((1, 1, RB), lambda i: (i, 0, 0)),
        out_shape=jax.ShapeDtypeStruct((NBLK, 1, RB), jnp.int32),
    )(action_values)
    return out.reshape(NW, CH, 128)


def _sc_body(actual_hbm, fidx_hbm, rew_hbm, done_hbm, out_hbm,
             idx_v, vals_v, rew_v, done_v, out_v, sem):
    wid = lax.axis_index("s") * NC + lax.axis_index("c")
    pltpu.sync_copy(fidx_hbm.at[wid], idx_v)
    pltpu.sync_copy(rew_hbm.at[wid], rew_v)
    pltpu.sync_copy(done_hbm.at[wid], done_v)
    descs = [
        pltpu.async_copy(actual_hbm.at[idx_v.at[j]], vals_v.at[j], sem)
        for j in range(CH)
    ]
    for d in descs:
        d.wait()
    for j in range(CH):
        for k in range(128 // L):
            sl = pl.ds(k * L, L)
            v = vals_v[j, sl]
            dn = done_v[j, sl]
            rw = rew_v[j, sl]
            w = jnp.where(dn != jnp.float32(0.0), jnp.float32(0.0), v)
            out_v[j, sl] = w * jnp.float32(GAMMA) + rw
    pltpu.sync_copy(out_v, out_hbm.at[wid])


def _sc_gather_epilogue(actual_flat, fidx, rew, done_f):
    mesh = plsc.VectorSubcoreMesh(
        core_axis_name="c", subcore_axis_name="s",
        num_cores=NC, num_subcores=NS,
    )
    f = functools.partial(
        pl.kernel,
        mesh=mesh,
        out_type=jax.ShapeDtypeStruct((NW, CH, 128), jnp.float32),
        scratch_types=[
            pltpu.VMEM((CH, 128), jnp.int32),
            pltpu.VMEM((CH, 128), jnp.float32),
            pltpu.VMEM((CH, 128), jnp.float32),
            pltpu.VMEM((CH, 128), jnp.float32),
            pltpu.VMEM((CH, 128), jnp.float32),
            pltpu.SemaphoreType.DMA,
        ],
    )(_sc_body)
    return f(actual_flat, fidx, rew, done_f)


def kernel(next_state_actual_values, next_state_action_values, reward, is_done):
    fidx = _argmax_flat(next_state_action_values)
    actual_flat = next_state_actual_values.reshape(B * A)
    rew = reward.reshape(NW, CH, 128)
    done_f = is_done.astype(jnp.float32).reshape(NW, CH, 128)
    out = _sc_gather_epilogue(actual_flat, fidx, rew, done_f)
    return out.reshape(B)


# speedup vs baseline: 1.2991x; 1.2991x over previous
import jax
import jax.numpy as jnp
from jax import lax
from jax.experimental import pallas as pl

GAMMA = 0.99

B = 16384
A = 1024
RB = 512
NBLK = B // RB


def _body(actual_ref, action_ref, rew_ref, done_ref, out_ref):
    av = action_ref[...]
    ac = actual_ref[...]
    mx = jnp.max(av, axis=1, keepdims=True)
    cols = lax.broadcasted_iota(jnp.int32, (RB, A), 1)
    big = jnp.int32(2**30)
    cstar = jnp.min(jnp.where(av == mx, cols, big), axis=1, keepdims=True)
    mask = cols == cstar
    val = jnp.sum(jnp.where(mask, ac, jnp.float32(0.0)), axis=1, keepdims=True)
    dn = done_ref[...]
    rw = rew_ref[...]
    w = jnp.where(dn != jnp.float32(0.0), jnp.float32(0.0), val)
    out_ref[...] = w * jnp.float32(GAMMA) + rw


def kernel(next_state_actual_values, next_state_action_values, reward, is_done):
    done_f = is_done.astype(jnp.float32)
    out = pl.pallas_call(
        _body,
        grid=(NBLK,),
        in_specs=[
            pl.BlockSpec((RB, A), lambda i: (i, 0)),
            pl.BlockSpec((RB, A), lambda i: (i, 0)),
            pl.BlockSpec((RB, 1), lambda i: (i, 0)),
            pl.BlockSpec((RB, 1), lambda i: (i, 0)),
        ],
        out_specs=pl.BlockSpec((RB, 1), lambda i: (i, 0)),
        out_shape=jax.ShapeDtypeStruct((B, 1), jnp.float32),
    )(next_state_actual_values, next_state_action_values, reward, done_f)
    return out.reshape(B)


# speedup vs baseline: 1.4044x; 1.0810x over previous
import jax
import jax.numpy as jnp
from jax import lax
from jax.experimental import pallas as pl

GAMMA = 0.99

B = 16384
A = 1024
RB = 1024
NBLK = B // RB


def _body(actual_ref, action_ref, rew_ref, done_ref, out_ref):
    av = action_ref[...]
    ac = actual_ref[...]
    mx = jnp.max(av, axis=1, keepdims=True)
    cols = lax.broadcasted_iota(jnp.int32, (RB, A), 1)
    big = jnp.int32(2**30)
    cstar = jnp.min(jnp.where(av == mx, cols, big), axis=1, keepdims=True)
    mask = cols == cstar
    val = jnp.sum(jnp.where(mask, ac, jnp.float32(0.0)), axis=1, keepdims=True)
    dn = done_ref[...]
    rw = rew_ref[...]
    w = jnp.where(dn != jnp.float32(0.0), jnp.float32(0.0), val)
    out_ref[...] = w * jnp.float32(GAMMA) + rw


def kernel(next_state_actual_values, next_state_action_values, reward, is_done):
    done_f = is_done.astype(jnp.float32)
    out = pl.pallas_call(
        _body,
        grid=(NBLK,),
        in_specs=[
            pl.BlockSpec((RB, A), lambda i: (i, 0)),
            pl.BlockSpec((RB, A), lambda i: (i, 0)),
            pl.BlockSpec((RB, 1), lambda i: (i, 0)),
            pl.BlockSpec((RB, 1), lambda i: (i, 0)),
        ],
        out_specs=pl.BlockSpec((RB, 1), lambda i: (i, 0)),
        out_shape=jax.ShapeDtypeStruct((B, 1), jnp.float32),
    )(next_state_actual_values, next_state_action_values, reward, done_f)
    return out.reshape(B)
